# baseline (speedup 1.0000x reference)
import functools

import jax
import jax.numpy as jnp
from jax import lax
from jax.experimental import pallas as pl
from jax.experimental.pallas import tpu as pltpu
from jax.experimental.pallas import tpu_sc as plsc

D = 64
SCALE = 8.0
NC = 2
NS = 16
L = 16
NW = NC * NS
CHUNK = 128


def _make_kernel(n_chunks: int, B: int):
  mesh = plsc.VectorSubcoreMesh(
      core_axis_name="c", subcore_axis_name="s",
      num_cores=NC, num_subcores=NS)

  @functools.partial(
      pl.kernel,
      mesh=mesh,
      out_type=jax.ShapeDtypeStruct((B, D), jnp.float32),
      scratch_types=[
          pltpu.VMEM((n_chunks, CHUNK), jnp.int32),
          pltpu.VMEM((CHUNK, D), jnp.float32),
          pltpu.SemaphoreType.DMA,
      ],
      compiler_params=pltpu.CompilerParams(use_tc_tiling_on_sc=False),
  )
  def emb(idx_hbm, table_hbm, out_hbm, idx_v, rows_v, sem):
    wid = lax.axis_index("s") * NC + lax.axis_index("c")
    base = wid * (n_chunks * CHUNK)
    pltpu.sync_copy(idx_hbm.at[wid], idx_v)

    def chunk_body(j, carry):
      pltpu.async_copy(table_hbm.at[idx_v.at[j]], rows_v, sem).wait()

      def scale_body(i, c):
        for k in range(D // L):
          rows_v[i, pl.ds(k * L, L)] = rows_v[i, pl.ds(k * L, L)] * SCALE
        return c

      lax.fori_loop(0, CHUNK, scale_body, 0)
      pltpu.sync_copy(rows_v, out_hbm.at[pl.ds(base + j * CHUNK, CHUNK)])
      return carry

    lax.fori_loop(0, n_chunks, chunk_body, 0)

  return emb


def kernel(inputs, table):
  rows, cols = inputs.shape
  B = rows * cols
  n_chunks = B // (NW * CHUNK)
  assert n_chunks * NW * CHUNK == B
  idx = inputs.reshape(NW, n_chunks, CHUNK)
  out = _make_kernel(n_chunks, B)(idx, table)
  return out.reshape(rows, cols, D)


# speedup vs baseline: 1.2078x; 1.2078x over previous
import functools

import jax
import jax.numpy as jnp
from jax import lax
from jax.experimental import pallas as pl
from jax.experimental.pallas import tpu as pltpu
from jax.experimental.pallas import tpu_sc as plsc

D = 64
SCALE = 8.0
NC = 2
NS = 16
L = 16
NW = NC * NS
CHUNK = 128
NBUF = 4


def _make_kernel(n_chunks: int, B: int):
  mesh = plsc.VectorSubcoreMesh(
      core_axis_name="c", subcore_axis_name="s",
      num_cores=NC, num_subcores=NS)
  n_groups = n_chunks // NBUF
  assert n_groups * NBUF == n_chunks

  scratch = (
      [pltpu.VMEM((n_chunks, CHUNK), jnp.int32)]
      + [pltpu.VMEM((CHUNK, D), jnp.float32) for _ in range(2 * NBUF)]
      + [pltpu.SemaphoreType.DMA for _ in range(2 * NBUF)]
  )

  @functools.partial(
      pl.kernel,
      mesh=mesh,
      out_type=jax.ShapeDtypeStruct((B, D), jnp.float32),
      scratch_types=scratch,
      compiler_params=pltpu.CompilerParams(use_tc_tiling_on_sc=False),
  )
  def emb(idx_hbm, table_hbm, out_hbm, idx_v, *rest):
    g_bufs = rest[0:NBUF]
    o_bufs = rest[NBUF:2 * NBUF]
    g_sems = rest[2 * NBUF:3 * NBUF]
    o_sems = rest[3 * NBUF:4 * NBUF]

    wid = lax.axis_index("s") * NC + lax.axis_index("c")
    base = wid * (n_chunks * CHUNK)
    pltpu.sync_copy(idx_hbm.at[wid], idx_v)

    for b in range(NBUF):
      pltpu.async_copy(table_hbm.at[idx_v.at[b]], g_bufs[b], g_sems[b])

    def group_body(g, carry):
      for b in range(NBUF):
        j = g * NBUF + b
        pltpu.make_async_copy(
            table_hbm.at[idx_v.at[b]], g_bufs[b], g_sems[b]).wait()

        @pl.when(g > 0)
        def _(b=b):
          pltpu.make_async_copy(
              o_bufs[b], out_hbm.at[pl.ds(base, CHUNK)], o_sems[b]).wait()

        def scale_body(i, b=b):
          for k in range(D // L):
            o_bufs[b][i, pl.ds(k * L, L)] = (
                g_bufs[b][i, pl.ds(k * L, L)] * SCALE)

        plsc.parallel_loop(0, CHUNK, unroll=8)(scale_body)

        pltpu.async_copy(
            o_bufs[b], out_hbm.at[pl.ds(base + j * CHUNK, CHUNK)], o_sems[b])

        @pl.when(g < n_groups - 1)
        def _(b=b, j=j):
          pltpu.async_copy(
              table_hbm.at[idx_v.at[j + NBUF]], g_bufs[b], g_sems[b])

      return carry

    lax.fori_loop(0, n_groups, group_body, 0)

    for b in range(NBUF):
      pltpu.make_async_copy(
          o_bufs[b], out_hbm.at[pl.ds(base, CHUNK)], o_sems[b]).wait()

  return emb


def kernel(inputs, table):
  rows, cols = inputs.shape
  B = rows * cols
  n_chunks = B // (NW * CHUNK)
  assert n_chunks * NW * CHUNK == B
  idx = inputs.reshape(NW, n_chunks, CHUNK)
  out = _make_kernel(n_chunks, B)(idx, table)
  return out.reshape(rows, cols, D)


# speedup vs baseline: 1.2084x; 1.0005x over previous
import functools

import jax
import jax.numpy as jnp
from jax import lax
from jax.experimental import pallas as pl
from jax.experimental.pallas import tpu as pltpu
from jax.experimental.pallas import tpu_sc as plsc

D = 64
SCALE = 8.0
NC = 2
NS = 16
L = 16
NW = NC * NS
CHUNK = 128
NBUF = 4


def _make_kernel(n_chunks: int, B: int):
  mesh = plsc.VectorSubcoreMesh(
      core_axis_name="c", subcore_axis_name="s",
      num_cores=NC, num_subcores=NS)
  n_groups = n_chunks // NBUF
  assert n_groups * NBUF == n_chunks

  scratch = (
      [pltpu.VMEM((n_chunks, CHUNK), jnp.int32)]
      + [pltpu.VMEM((CHUNK, D), jnp.float32) for _ in range(2 * NBUF)]
      + [pltpu.SemaphoreType.DMA for _ in range(2 * NBUF)]
  )

  @functools.partial(
      pl.kernel,
      mesh=mesh,
      out_type=jax.ShapeDtypeStruct((B, D), jnp.float32),
      scratch_types=scratch,
      compiler_params=pltpu.CompilerParams(
          use_tc_tiling_on_sc=False, needs_layout_passes=True),
  )
  def emb(idx_hbm, table_hbm, out_hbm, idx_v, *rest):
    g_bufs = rest[0:NBUF]
    o_bufs = rest[NBUF:2 * NBUF]
    g_sems = rest[2 * NBUF:3 * NBUF]
    o_sems = rest[3 * NBUF:4 * NBUF]

    wid = lax.axis_index("s") * NC + lax.axis_index("c")
    base = wid * (n_chunks * CHUNK)
    pltpu.sync_copy(idx_hbm.at[wid], idx_v)

    for b in range(NBUF):
      pltpu.async_copy(table_hbm.at[idx_v.at[b]], g_bufs[b], g_sems[b])

    def group_body(g, carry):
      for b in range(NBUF):
        j = g * NBUF + b
        pltpu.make_async_copy(
            table_hbm.at[idx_v.at[b]], g_bufs[b], g_sems[b]).wait()

        @pl.when(g > 0)
        def _(b=b):
          pltpu.make_async_copy(
              o_bufs[b], out_hbm.at[pl.ds(base, CHUNK)], o_sems[b]).wait()

        def scale_body(i, b=b):
          for k in range(D // L):
            o_bufs[b][i, pl.ds(k * L, L)] = (
                g_bufs[b][i, pl.ds(k * L, L)] * SCALE)

        plsc.parallel_loop(0, CHUNK, unroll=8)(scale_body)

        pltpu.async_copy(
            o_bufs[b], out_hbm.at[pl.ds(base + j * CHUNK, CHUNK)], o_sems[b])

        @pl.when(g < n_groups - 1)
        def _(b=b, j=j):
          pltpu.async_copy(
              table_hbm.at[idx_v.at[j + NBUF]], g_bufs[b], g_sems[b])

      return carry

    lax.fori_loop(0, n_groups, group_body, 0)

    for b in range(NBUF):
      pltpu.make_async_copy(
          o_bufs[b], out_hbm.at[pl.ds(base, CHUNK)], o_sems[b]).wait()

  return emb


def kernel(inputs, table):
  rows, cols = inputs.shape
  B = rows * cols
  n_chunks = B // (NW * CHUNK)
  assert n_chunks * NW * CHUNK == B
  idx = inputs.reshape(NW, n_chunks, CHUNK)
  out = _make_kernel(n_chunks, B)(idx, table)
  return out.reshape(rows, cols, D)
